# Initial kernel scaffold; baseline (speedup 1.0000x reference)
import jax
import jax.numpy as jnp
from jax.experimental import pallas as pl


def kernel(x, edge_index, edge_attr, batch, Wpre, bpre, Wedge, bedge, Wpost, bpost, Wlin, blin, bn_gamma, bn_beta, W1, b1, W2, b2, W3, b3):
    raise NotImplementedError("write your pallas kernel here")


# speedup vs baseline: 33.2875x; 33.2875x over previous
import numpy as np
import jax
import jax.numpy as jnp
from jax.experimental import pallas as pl

_DEG = np.array([0, 0, 0, 0, 0, 0, 200, 400, 800, 1200, 1800, 2400, 3000,
                 3600, 4000, 4300, 4400, 4400, 4300, 4000, 3600, 3000, 2400,
                 1800, 1200, 800, 400, 200], dtype=np.float64)
_AVG_LOG = float((np.log(np.arange(_DEG.shape[0]) + 1.0) * _DEG).sum() / _DEG.sum())

_TILE = 2000
_NUM_GRAPHS = 512


def _prep_body(h_ref, a_ref, b_ref, c_ref, p_ref, hs_ref):
    h = h_ref[...]
    p_ref[...] = jnp.dot(h, a_ref[...], preferred_element_type=jnp.float32) + c_ref[...]
    hs_ref[...] = jnp.dot(h, b_ref[...], preferred_element_type=jnp.float32)


def _node_body(h_ref, p_ref, cnt_ref, s1_ref, s2_ref, mn_ref, mx_ref,
               wh_ref, wg1_ref, wg2_ref, wg3_ref, bp_ref, wlin_ref, blin_ref,
               out_ref, stats_ref):
    h = h_ref[...]
    p = p_ref[...]
    cnt = cnt_ref[...]
    s1 = s1_ref[...]
    s2 = s2_ref[...]
    cntc = jnp.maximum(cnt, 1.0)
    inv = 1.0 / cntc
    mean = (cnt * p + s1) * inv
    has = cnt > 0
    mnm = jnp.where(has, p + mn_ref[...], 0.0)
    mxm = jnp.where(has, p + mx_ref[...], 0.0)
    msq = (cnt * p * p + 2.0 * p * s1 + s2) * inv
    std = jnp.sqrt(jax.nn.relu(msq - mean * mean) + 1e-5)
    lg = jnp.log(cntc + 1.0)
    s1c = lg / _AVG_LOG
    s2c = _AVG_LOG / lg
    agg4 = jnp.concatenate([mean, mnm, mxm, std], axis=1)
    g1 = jnp.dot(agg4, wg1_ref[...], preferred_element_type=jnp.float32)
    g2 = jnp.dot(agg4, wg2_ref[...], preferred_element_type=jnp.float32)
    g3 = jnp.dot(agg4, wg3_ref[...], preferred_element_type=jnp.float32)
    res = (jnp.dot(h, wh_ref[...], preferred_element_type=jnp.float32)
           + g1 + s1c * g2 + s2c * g3 + bp_ref[...])
    out5 = jnp.dot(res, wlin_ref[...], preferred_element_type=jnp.float32) + blin_ref[...]
    out_ref[...] = out5
    s = jnp.sum(out5, axis=0, keepdims=True)
    ss = jnp.sum(out5 * out5, axis=0, keepdims=True)
    st = jnp.pad(jnp.concatenate([s, ss], axis=0), ((0, 6), (0, 123)))

    @pl.when(pl.program_id(0) == 0)
    def _():
        stats_ref[...] = st

    @pl.when(pl.program_id(0) > 0)
    def _():
        stats_ref[...] += st


def _bn_body(out_ref, stats_ref, g_ref, b_ref, h_ref):
    nf = jnp.float32(50000.0)
    mu = stats_ref[0:1, 0:5] / nf
    var = stats_ref[1:2, 0:5] / nf - mu * mu
    y = (out_ref[...] - mu) / jnp.sqrt(var + 1e-5) * g_ref[...] + b_ref[...]
    h_ref[...] = jnp.maximum(y, 0.0)


def _bn_pool_body(out_ref, stats_ref, g_ref, b_ref, batch_ref, h_ref, pool_ref):
    nf = jnp.float32(50000.0)
    mu = stats_ref[0:1, 0:5] / nf
    var = stats_ref[1:2, 0:5] / nf - mu * mu
    y = (out_ref[...] - mu) / jnp.sqrt(var + 1e-5) * g_ref[...] + b_ref[...]
    hn = jnp.maximum(y, 0.0)
    h_ref[...] = hn
    io = jax.lax.broadcasted_iota(jnp.int32, (_TILE, _NUM_GRAPHS), 1)
    oh = (batch_ref[...] == io).astype(jnp.float32)
    part = jax.lax.dot_general(oh, hn, (((0,), (0,)), ((), ())),
                               preferred_element_type=jnp.float32)

    @pl.when(pl.program_id(0) == 0)
    def _():
        pool_ref[...] = part

    @pl.when(pl.program_id(0) > 0)
    def _():
        pool_ref[...] += part


def _mlp_body(p_ref, w1_ref, b1_ref, w2_ref, b2_ref, w3_ref, b3_ref, o_ref):
    z = jnp.maximum(jnp.dot(p_ref[...], w1_ref[...],
                            preferred_element_type=jnp.float32) + b1_ref[...], 0.0)
    z = jnp.maximum(jnp.dot(z, w2_ref[...],
                            preferred_element_type=jnp.float32) + b2_ref[...], 0.0)
    o_ref[...] = jnp.dot(z, w3_ref[...],
                         preferred_element_type=jnp.float32) + b3_ref[...]


def _gmat(wp, base):
    blocks = []
    eye = jnp.eye(5, dtype=jnp.float32)
    for a in range(4):
        sub = wp[:, base + a * 5: base + a * 5 + 5]
        blocks.append((eye[:, None, :] * sub[:, :, None]).reshape(25, 5))
    return jnp.concatenate(blocks, axis=0)


def kernel(x, edge_index, edge_attr, batch, Wpre, bpre, Wedge, bedge, Wpost,
           bpost, Wlin, blin, bn_gamma, bn_beta, W1, b1, W2, b2, W3, b3):
    n = x.shape[0]
    nt = n // _TILE
    src = edge_index[0]
    dst = edge_index[1]
    h = x
    pooled = None
    for i in range(2):
        wpre = Wpre[i]
        pa = wpre[:, 0:5, :].transpose(1, 0, 2).reshape(5, 25)
        pb = wpre[:, 5:10, :].transpose(1, 0, 2).reshape(5, 25)
        pc = wpre[:, 10:15, :].transpose(1, 0, 2).reshape(5, 25)
        wvec = (Wedge[i][0] @ pc)
        cvec = (bedge[i] @ pc + bpre[i].reshape(25))[None, :]

        p_arr, hs = pl.pallas_call(
            _prep_body,
            grid=(nt,),
            in_specs=[
                pl.BlockSpec((_TILE, 5), lambda j: (j, 0)),
                pl.BlockSpec((5, 25), lambda j: (0, 0)),
                pl.BlockSpec((5, 25), lambda j: (0, 0)),
                pl.BlockSpec((1, 25), lambda j: (0, 0)),
            ],
            out_specs=[
                pl.BlockSpec((_TILE, 25), lambda j: (j, 0)),
                pl.BlockSpec((_TILE, 25), lambda j: (j, 0)),
            ],
            out_shape=[
                jax.ShapeDtypeStruct((n, 25), jnp.float32),
                jax.ShapeDtypeStruct((n, 25), jnp.float32),
            ],
        )(h, pa, pb, cvec)

        q = jnp.take(hs, src, axis=0) + edge_attr * wvec[None, :]
        ones = jnp.ones((src.shape[0],), jnp.float32)
        cnt = jax.ops.segment_sum(ones, dst, num_segments=n).reshape(n, 1)
        s1 = jax.ops.segment_sum(q, dst, num_segments=n)
        s2 = jax.ops.segment_sum(q * q, dst, num_segments=n)
        mn = jax.ops.segment_min(q, dst, num_segments=n)
        mx = jax.ops.segment_max(q, dst, num_segments=n)

        wp = Wpost[i][:, :, 0]
        wh = wp[:, 0:5].T
        wg1 = _gmat(wp, 5)
        wg2 = _gmat(wp, 25)
        wg3 = _gmat(wp, 45)
        bp = bpost[i][:, 0][None, :]

        out5, stats = pl.pallas_call(
            _node_body,
            grid=(nt,),
            in_specs=[
                pl.BlockSpec((_TILE, 5), lambda j: (j, 0)),
                pl.BlockSpec((_TILE, 25), lambda j: (j, 0)),
                pl.BlockSpec((_TILE, 1), lambda j: (j, 0)),
                pl.BlockSpec((_TILE, 25), lambda j: (j, 0)),
                pl.BlockSpec((_TILE, 25), lambda j: (j, 0)),
                pl.BlockSpec((_TILE, 25), lambda j: (j, 0)),
                pl.BlockSpec((_TILE, 25), lambda j: (j, 0)),
                pl.BlockSpec((5, 5), lambda j: (0, 0)),
                pl.BlockSpec((100, 5), lambda j: (0, 0)),
                pl.BlockSpec((100, 5), lambda j: (0, 0)),
                pl.BlockSpec((100, 5), lambda j: (0, 0)),
                pl.BlockSpec((1, 5), lambda j: (0, 0)),
                pl.BlockSpec((5, 5), lambda j: (0, 0)),
                pl.BlockSpec((1, 5), lambda j: (0, 0)),
            ],
            out_specs=[
                pl.BlockSpec((_TILE, 5), lambda j: (j, 0)),
                pl.BlockSpec((8, 128), lambda j: (0, 0)),
            ],
            out_shape=[
                jax.ShapeDtypeStruct((n, 5), jnp.float32),
                jax.ShapeDtypeStruct((8, 128), jnp.float32),
            ],
        )(h, p_arr, cnt, s1, s2, mn, mx, wh, wg1, wg2, wg3, bp, Wlin[i],
          blin[i][None, :])

        gamma = bn_gamma[i][None, :]
        beta = bn_beta[i][None, :]
        if i == 0:
            h = pl.pallas_call(
                _bn_body,
                grid=(nt,),
                in_specs=[
                    pl.BlockSpec((_TILE, 5), lambda j: (j, 0)),
                    pl.BlockSpec((8, 128), lambda j: (0, 0)),
                    pl.BlockSpec((1, 5), lambda j: (0, 0)),
                    pl.BlockSpec((1, 5), lambda j: (0, 0)),
                ],
                out_specs=pl.BlockSpec((_TILE, 5), lambda j: (j, 0)),
                out_shape=jax.ShapeDtypeStruct((n, 5), jnp.float32),
            )(out5, stats, gamma, beta)
        else:
            h, pooled = pl.pallas_call(
                _bn_pool_body,
                grid=(nt,),
                in_specs=[
                    pl.BlockSpec((_TILE, 5), lambda j: (j, 0)),
                    pl.BlockSpec((8, 128), lambda j: (0, 0)),
                    pl.BlockSpec((1, 5), lambda j: (0, 0)),
                    pl.BlockSpec((1, 5), lambda j: (0, 0)),
                    pl.BlockSpec((_TILE, 1), lambda j: (j, 0)),
                ],
                out_specs=[
                    pl.BlockSpec((_TILE, 5), lambda j: (j, 0)),
                    pl.BlockSpec((_NUM_GRAPHS, 5), lambda j: (0, 0)),
                ],
                out_shape=[
                    jax.ShapeDtypeStruct((n, 5), jnp.float32),
                    jax.ShapeDtypeStruct((_NUM_GRAPHS, 5), jnp.float32),
                ],
            )(out5, stats, gamma, beta, batch.reshape(n, 1))

    return pl.pallas_call(
        _mlp_body,
        in_specs=[
            pl.BlockSpec((_NUM_GRAPHS, 5), lambda: (0, 0)),
            pl.BlockSpec((5, 5), lambda: (0, 0)),
            pl.BlockSpec((1, 5), lambda: (0, 0)),
            pl.BlockSpec((5, 10), lambda: (0, 0)),
            pl.BlockSpec((1, 10), lambda: (0, 0)),
            pl.BlockSpec((10, 10), lambda: (0, 0)),
            pl.BlockSpec((1, 10), lambda: (0, 0)),
        ],
        out_specs=pl.BlockSpec((_NUM_GRAPHS, 10), lambda: (0, 0)),
        out_shape=jax.ShapeDtypeStruct((_NUM_GRAPHS, 10), jnp.float32),
    )(pooled, W1, b1[None, :], W2, b2[None, :], W3, b3[None, :])
